# baseline (speedup 1.0000x reference)
import functools

import jax
import jax.numpy as jnp
from jax.experimental import pallas as pl
from jax.experimental.pallas import tpu as pltpu
from jax.experimental.pallas import tpu_sc as plsc

EPS = 1e-6
KNN = 16
BIGF = 3.0e38


def _sc_gather_kernel(window, data_hbm, idx_hbm, out_hbm):
    num = idx_hbm.shape[1]

    def body(i_vmem, o_vmem):
        pltpu.sync_copy(data_hbm.at[i_vmem.at[0]], o_vmem)

    pltpu.emit_pipeline(
        body,
        grid=(num // window,),
        in_specs=[pl.BlockSpec((1, window), index_map=lambda i: (0, i))],
        out_specs=[pl.BlockSpec((window, data_hbm.shape[1]),
                                index_map=lambda i: (i, 0))],
        core_axis_name=("c", "s"),
        dimension_semantics=(pltpu.PARALLEL,),
    )(idx_hbm, out_hbm)


def sc_gather(data2d, idx_flat):
    num = idx_flat.shape[0]
    window = 128
    quantum = window * 32
    orig = num
    if num % quantum:
        pad = quantum - num % quantum
        filler = jnp.arange(pad, dtype=idx_flat.dtype) % data2d.shape[0]
        idx_flat = jnp.concatenate([idx_flat, filler])
        num = num + pad
    idx2 = idx_flat.reshape(1, num)
    out_t = jax.ShapeDtypeStruct((num, data2d.shape[1]), data2d.dtype)
    mesh = plsc.VectorSubcoreMesh(core_axis_name="c", subcore_axis_name="s")
    k = pl.kernel(functools.partial(_sc_gather_kernel, window),
                  out_type=out_t, mesh=mesh)
    out = k(data2d, idx2)
    return out[:orig] if orig != num else out


def _knn_body(N, QB, K, xq_ref, xt_ref, idx_ref):
    b = pl.program_id(0)
    xq = xq_ref[0]
    xtT = xt_ref[0]
    inner = jax.lax.dot_general(
        xq.astype(jnp.bfloat16), xtT.astype(jnp.bfloat16),
        (((1,), (0,)), ((), ())), preferred_element_type=jnp.float32)
    sqr = jnp.sum(xtT * xtT, axis=0, keepdims=True)
    dist = sqr - 2.0 * inner
    lane = jax.lax.broadcasted_iota(jnp.int32, (QB, N), 1)
    kcol = jax.lax.broadcasted_iota(jnp.int32, (QB, K), 1)
    acc = jnp.zeros((QB, K), jnp.int32)
    for j in range(K):
        am = jnp.argmin(dist, axis=1).astype(jnp.int32)[:, None]
        acc = jnp.where(kcol == j, am, acc)
        dist = jnp.where(lane == am, BIGF, dist)
    idx_ref[0] = acc + b * N


def knn_call(xrows, QB):
    B, N, F = xrows.shape
    grid = (B, N // QB)
    return pl.pallas_call(
        functools.partial(_knn_body, N, QB, KNN),
        grid=grid,
        in_specs=[
            pl.BlockSpec((1, QB, F), lambda b, q: (b, q, 0)),
            pl.BlockSpec((1, F, N), lambda b, q: (b, 0, 0)),
        ],
        out_specs=pl.BlockSpec((1, QB, KNN), lambda b, q: (b, q, 0)),
        out_shape=jax.ShapeDtypeStruct((B, N, KNN), jnp.int32),
        compiler_params=pltpu.CompilerParams(
            dimension_semantics=("arbitrary", "arbitrary")),
    )(xrows, xrows.transpose(0, 2, 1))


def _vn_edge(nbr, xv, C, QB, K):
    rep = jnp.broadcast_to(xv[:, None, :], (QB, K, C)).reshape(QB * K, C)
    return jnp.concatenate([nbr - rep, rep], axis=1)


def _bf_dot(a, w):
    return jax.lax.dot_general(
        a.astype(jnp.bfloat16), w.astype(jnp.bfloat16),
        (((1,), (0,)), ((), ())), preferred_element_type=jnp.float32)


def _stats_body(C, Co, QB, K, F, nbr_ref, x_ref, wf_ref, out_ref):
    first = (pl.program_id(0) == 0) & (pl.program_id(1) == 0)

    @pl.when(first)
    def _():
        out_ref[...] = jnp.zeros_like(out_ref)

    wf = wf_ref[...]
    normsq = jnp.zeros((QB * K, Co), jnp.float32)
    for v in range(3):
        nv = nbr_ref[0][:, v * C:(v + 1) * C]
        xv = x_ref[0][:, v * C:(v + 1) * C]
        p = _bf_dot(_vn_edge(nv, xv, C, QB, K), wf)
        normsq = normsq + p * p
    norm = jnp.sqrt(normsq) + EPS
    s1 = jnp.sum(norm, axis=0, keepdims=True)
    s2 = jnp.sum(norm * norm, axis=0, keepdims=True)
    out_ref[...] += jnp.concatenate([s1, s2], axis=0)


def _apply_body(C, Co, QB, K, F, cnt, opad, nbr_ref, x_ref, wf_ref,
                wd_ref, g_ref, b_ref, st_ref, out_ref):
    wf = wf_ref[...]
    wd = wd_ref[...]
    ps = []
    ds = []
    for v in range(3):
        nv = nbr_ref[0][:, v * C:(v + 1) * C]
        xv = x_ref[0][:, v * C:(v + 1) * C]
        edge = _vn_edge(nv, xv, C, QB, K)
        ps.append(_bf_dot(edge, wf))
        ds.append(_bf_dot(edge, wd))
    normsq = ps[0] * ps[0] + ps[1] * ps[1] + ps[2] * ps[2]
    norm = jnp.sqrt(normsq) + EPS
    mean = st_ref[0:1, :] * (1.0 / cnt)
    var = st_ref[1:2, :] * (1.0 / cnt) - mean * mean
    inv = 1.0 / jnp.sqrt(var + 1e-5)
    norm_bn = g_ref[...] * (norm - mean) * inv + b_ref[...]
    factor = norm_bn / norm
    psc = [p * factor for p in ps]
    dot = psc[0] * ds[0] + psc[1] * ds[1] + psc[2] * ds[2]
    dnsq = ds[0] * ds[0] + ds[1] * ds[1] + ds[2] * ds[2]
    coef = jnp.where(dot < 0, 0.8 * dot / (dnsq + EPS), 0.0)
    outs = []
    for v in range(3):
        ov = psc[v] - coef * ds[v]
        outs.append(jnp.mean(ov.reshape(QB, K, Co), axis=1))
    if opad:
        outs.append(jnp.zeros((QB, opad), jnp.float32))
    out_ref[0] = jnp.concatenate(outs, axis=1)


def vn_block(nbr, xrows, Wf, Wd, gamma, beta, C, Co, QB, opad=0):
    B, N, F = xrows.shape
    K = KNN
    cnt = float(B * N * K)
    wft = Wf.T
    wdt = Wd.T
    g2 = gamma.reshape(1, Co)
    b2 = beta.reshape(1, Co)
    grid = (B, N // QB)
    Fp = nbr.shape[-1]
    nbr3 = nbr.reshape(B, N * K, Fp)
    nbr_spec = pl.BlockSpec((1, QB * K, Fp), lambda b, q: (b, q, 0))
    x_spec = pl.BlockSpec((1, QB, F), lambda b, q: (b, q, 0))
    w_spec = pl.BlockSpec((2 * C, Co), lambda b, q: (0, 0))
    v_spec = pl.BlockSpec((1, Co), lambda b, q: (0, 0))
    st = pl.pallas_call(
        functools.partial(_stats_body, C, Co, QB, K, F),
        grid=grid,
        in_specs=[nbr_spec, x_spec, w_spec],
        out_specs=pl.BlockSpec((2, Co), lambda b, q: (0, 0)),
        out_shape=jax.ShapeDtypeStruct((2, Co), jnp.float32),
        compiler_params=pltpu.CompilerParams(
            dimension_semantics=("arbitrary", "arbitrary")),
    )(nbr3, xrows, wft)
    W = 3 * Co + opad
    out = pl.pallas_call(
        functools.partial(_apply_body, C, Co, QB, K, F, cnt, opad),
        grid=grid,
        in_specs=[nbr_spec, x_spec, w_spec, w_spec,
                  v_spec, v_spec, pl.BlockSpec((2, Co), lambda b, q: (0, 0))],
        out_specs=pl.BlockSpec((1, QB, W), lambda b, q: (b, q, 0)),
        out_shape=jax.ShapeDtypeStruct((B, N, W), jnp.float32),
        compiler_params=pltpu.CompilerParams(
            dimension_semantics=("arbitrary", "arbitrary")),
    )(nbr3, xrows, wft, wdt, g2, b2, st)
    return out


def _fps_body(B, N, M, coor_ref, idx_ref, nc_ref):
    cc = coor_ref[...].reshape(3 * B, N)
    c0 = cc[:B]
    c1 = cc[B:2 * B]
    c2 = cc[2 * B:]
    lane_n = jax.lax.broadcasted_iota(jnp.int32, (B, N), 1)
    lane_m = jax.lax.broadcasted_iota(jnp.int32, (B, M), 1)
    lane_m3 = jax.lax.broadcasted_iota(jnp.int32, (3 * B, M), 1)
    row_n = jax.lax.broadcasted_iota(jnp.int32, (B, N), 0)
    row_m = jax.lax.broadcasted_iota(jnp.int32, (B, M), 0)
    row_m3 = jax.lax.broadcasted_iota(jnp.int32, (3 * B, M), 0)

    def body(i, carry):
        dists, acc, nc = carry
        mx = jnp.max(dists, axis=1, keepdims=True)
        am = jnp.min(jnp.where(dists == mx, lane_n, N), axis=1, keepdims=True)
        far = jnp.where(i == 0, 0, am)
        mask = lane_n == far
        e0 = jnp.sum(jnp.where(mask, c0, 0.0), axis=1, keepdims=True)
        e1 = jnp.sum(jnp.where(mask, c1, 0.0), axis=1, keepdims=True)
        e2 = jnp.sum(jnp.where(mask, c2, 0.0), axis=1, keepdims=True)
        e = jnp.concatenate([e0, e1, e2], axis=0)
        acc = jnp.where(lane_m == i, jnp.broadcast_to(far, (B, M)), acc)
        nc = jnp.where(lane_m3 == i, jnp.broadcast_to(e, (3 * B, M)), nc)
        d = (c0 - e0) ** 2 + (c1 - e1) ** 2 + (c2 - e2) ** 2
        dists = jnp.minimum(dists, d)
        return dists, acc, nc

    init = (jnp.where(lane_n + row_n < 0, 0.0, 1e10),
            lane_m + row_m,
            (lane_m3 + row_m3).astype(jnp.float32))
    _, acc, nc = jax.lax.fori_loop(0, M, body, init)
    idx_ref[...] = acc + row_m * N
    nc_ref[...] = nc.reshape(3, B, M)


def fps_call(coor3, M):
    _, B, N = coor3.shape
    return pl.pallas_call(
        functools.partial(_fps_body, B, N, M),
        in_specs=[pl.BlockSpec((3, B, N), lambda: (0, 0, 0))],
        out_specs=[pl.BlockSpec((B, M), lambda: (0, 0)),
                   pl.BlockSpec((3, B, M), lambda: (0, 0, 0))],
        out_shape=[jax.ShapeDtypeStruct((B, M), jnp.int32),
                   jax.ShapeDtypeStruct((3, B, M), jnp.float32)],
    )(coor3)


def kernel(x, W1f, W1d, g1, b1, W4f, W4d, g4, b4, W5f, W5d, g5, b5,
           W6f, W6d, g6, b6):
    B, _, N1 = x.shape
    K = KNN
    xrows = x.transpose(0, 2, 1)
    coor3 = x.transpose(1, 0, 2)

    idx1 = knn_call(xrows, QB=512)
    xpad = jnp.pad(xrows.reshape(B * N1, 3), ((0, 0), (0, 125)))
    nbr1 = sc_gather(xpad, idx1.reshape(-1))
    out1 = vn_block(nbr1, xrows, W1f, W1d, g1, b1, C=1, Co=32, QB=256,
                    opad=32)

    idxf1, ncoor1 = fps_call(coor3, 512)
    fq1 = sc_gather(out1.reshape(B * N1, 128), idxf1.reshape(-1))
    fq1 = fq1.reshape(B, 512, 128)

    idx2 = knn_call(fq1, QB=512)
    nbr2 = sc_gather(fq1.reshape(B * 512, 128), idx2.reshape(-1))
    out2 = vn_block(nbr2, fq1, W4f, W4d, g4, b4, C=32, Co=64, QB=128,
                    opad=64)

    idx3 = knn_call(out2, QB=512)
    nbr3 = sc_gather(out2.reshape(B * 512, 256), idx3.reshape(-1))
    out3 = vn_block(nbr3, out2, W5f, W5d, g5, b5, C=64, Co=64, QB=128,
                    opad=64)

    idxf2, ncoor2 = fps_call(ncoor1, 128)
    fq2 = sc_gather(out3.reshape(B * 512, 256), idxf2.reshape(-1))
    fq2 = fq2.reshape(B, 128, 256)

    idx4 = knn_call(fq2, QB=128)
    nbr4 = sc_gather(fq2.reshape(B * 128, 256), idx4.reshape(-1))
    out4 = vn_block(nbr4, fq2, W6f, W6d, g6, b6, C=64, Co=128, QB=128)

    coor_out = ncoor2.transpose(1, 0, 2)
    f_out = out4.reshape(B, 128, 3, 128).transpose(0, 3, 2, 1)
    return coor_out, f_out


# speedup vs baseline: 1.0159x; 1.0159x over previous
import functools

import jax
import jax.numpy as jnp
from jax.experimental import pallas as pl
from jax.experimental.pallas import tpu as pltpu
from jax.experimental.pallas import tpu_sc as plsc

EPS = 1e-6
KNN = 16
BIGF = 3.0e38


def _sc_gather_kernel(window, data_hbm, idx_hbm, out_hbm):
    num = idx_hbm.shape[1]

    def body(i_vmem, o_vmem):
        pltpu.sync_copy(data_hbm.at[i_vmem.at[0]], o_vmem)

    pltpu.emit_pipeline(
        body,
        grid=(num // window,),
        in_specs=[pl.BlockSpec((1, window), index_map=lambda i: (0, i))],
        out_specs=[pl.BlockSpec((window, data_hbm.shape[1]),
                                index_map=lambda i: (i, 0))],
        core_axis_name=("c", "s"),
        dimension_semantics=(pltpu.PARALLEL,),
    )(idx_hbm, out_hbm)


def sc_gather(data2d, idx_flat):
    num = idx_flat.shape[0]
    window = 128
    quantum = window * 32
    orig = num
    if num % quantum:
        pad = quantum - num % quantum
        filler = jnp.arange(pad, dtype=idx_flat.dtype) % data2d.shape[0]
        idx_flat = jnp.concatenate([idx_flat, filler])
        num = num + pad
    idx2 = idx_flat.reshape(1, num)
    out_t = jax.ShapeDtypeStruct((num, data2d.shape[1]), data2d.dtype)
    mesh = plsc.VectorSubcoreMesh(core_axis_name="c", subcore_axis_name="s")
    k = pl.kernel(functools.partial(_sc_gather_kernel, window),
                  out_type=out_t, mesh=mesh)
    out = k(data2d, idx2)
    return out[:orig] if orig != num else out


def _knn_body(N, QB, K, xq_ref, xt_ref, idx_ref):
    b = pl.program_id(0)
    xq = xq_ref[0]
    xtT = xt_ref[0]
    inner = jax.lax.dot_general(
        xq.astype(jnp.bfloat16), xtT.astype(jnp.bfloat16),
        (((1,), (0,)), ((), ())), preferred_element_type=jnp.float32)
    sqr = jnp.sum(xtT * xtT, axis=0, keepdims=True)
    dist = sqr - 2.0 * inner
    lane = jax.lax.broadcasted_iota(jnp.int32, (QB, N), 1)
    kcol = jax.lax.broadcasted_iota(jnp.int32, (QB, K), 1)
    acc = jnp.zeros((QB, K), jnp.int32)
    for j in range(K):
        am = jnp.argmin(dist, axis=1).astype(jnp.int32)[:, None]
        acc = jnp.where(kcol == j, am, acc)
        dist = jnp.where(lane == am, BIGF, dist)
    idx_ref[0] = acc + b * N


def knn_call(xrows, QB):
    B, N, F = xrows.shape
    grid = (B, N // QB)
    return pl.pallas_call(
        functools.partial(_knn_body, N, QB, KNN),
        grid=grid,
        in_specs=[
            pl.BlockSpec((1, QB, F), lambda b, q: (b, q, 0)),
            pl.BlockSpec((1, F, N), lambda b, q: (b, 0, 0)),
        ],
        out_specs=pl.BlockSpec((1, QB, KNN), lambda b, q: (b, q, 0)),
        out_shape=jax.ShapeDtypeStruct((B, N, KNN), jnp.int32),
        compiler_params=pltpu.CompilerParams(
            dimension_semantics=("arbitrary", "arbitrary")),
    )(xrows, xrows.transpose(0, 2, 1))


def _vn_edge(nbr, xv, C, QB, K):
    rep = jnp.broadcast_to(xv[:, None, :], (QB, K, C)).reshape(QB * K, C)
    return jnp.concatenate([nbr - rep, rep], axis=1)


def _bf_dot(a, w):
    return jax.lax.dot_general(
        a.astype(jnp.bfloat16), w.astype(jnp.bfloat16),
        (((1,), (0,)), ((), ())), preferred_element_type=jnp.float32)


def _stats_body(C, Co, QB, K, F, nbr_ref, x_ref, wf_ref, out_ref):
    first = (pl.program_id(0) == 0) & (pl.program_id(1) == 0)

    @pl.when(first)
    def _():
        out_ref[...] = jnp.zeros_like(out_ref)

    wf = wf_ref[...]
    normsq = jnp.zeros((QB * K, Co), jnp.float32)
    for v in range(3):
        nv = nbr_ref[0][:, v * C:(v + 1) * C]
        xv = x_ref[0][:, v * C:(v + 1) * C]
        p = _bf_dot(_vn_edge(nv, xv, C, QB, K), wf)
        normsq = normsq + p * p
    norm = jnp.sqrt(normsq) + EPS
    s1 = jnp.sum(norm, axis=0, keepdims=True)
    s2 = jnp.sum(norm * norm, axis=0, keepdims=True)
    out_ref[...] += jnp.concatenate([s1, s2], axis=0)


def _apply_body(C, Co, QB, K, F, cnt, opad, nbr_ref, x_ref, wf_ref,
                wd_ref, g_ref, b_ref, st_ref, out_ref):
    wf = wf_ref[...]
    wd = wd_ref[...]
    ps = []
    ds = []
    for v in range(3):
        nv = nbr_ref[0][:, v * C:(v + 1) * C]
        xv = x_ref[0][:, v * C:(v + 1) * C]
        edge = _vn_edge(nv, xv, C, QB, K)
        ps.append(_bf_dot(edge, wf))
        ds.append(_bf_dot(edge, wd))
    normsq = ps[0] * ps[0] + ps[1] * ps[1] + ps[2] * ps[2]
    norm = jnp.sqrt(normsq) + EPS
    mean = st_ref[0:1, :] * (1.0 / cnt)
    var = st_ref[1:2, :] * (1.0 / cnt) - mean * mean
    inv = 1.0 / jnp.sqrt(var + 1e-5)
    norm_bn = g_ref[...] * (norm - mean) * inv + b_ref[...]
    factor = norm_bn / norm
    psc = [p * factor for p in ps]
    dot = psc[0] * ds[0] + psc[1] * ds[1] + psc[2] * ds[2]
    dnsq = ds[0] * ds[0] + ds[1] * ds[1] + ds[2] * ds[2]
    coef = jnp.where(dot < 0, 0.8 * dot / (dnsq + EPS), 0.0)
    outs = []
    for v in range(3):
        ov = psc[v] - coef * ds[v]
        outs.append(jnp.mean(ov.reshape(QB, K, Co), axis=1))
    if opad:
        outs.append(jnp.zeros((QB, opad), jnp.float32))
    out_ref[0] = jnp.concatenate(outs, axis=1)


def vn_block(nbr, xrows, Wf, Wd, gamma, beta, C, Co, QB, opad=0):
    B, N, F = xrows.shape
    K = KNN
    cnt = float(B * N * K)
    wft = Wf.T
    wdt = Wd.T
    g2 = gamma.reshape(1, Co)
    b2 = beta.reshape(1, Co)
    grid = (B, N // QB)
    Fp = nbr.shape[-1]
    nbr3 = nbr.reshape(B, N * K, Fp)
    nbr_spec = pl.BlockSpec((1, QB * K, Fp), lambda b, q: (b, q, 0))
    x_spec = pl.BlockSpec((1, QB, F), lambda b, q: (b, q, 0))
    w_spec = pl.BlockSpec((2 * C, Co), lambda b, q: (0, 0))
    v_spec = pl.BlockSpec((1, Co), lambda b, q: (0, 0))
    st = pl.pallas_call(
        functools.partial(_stats_body, C, Co, QB, K, F),
        grid=grid,
        in_specs=[nbr_spec, x_spec, w_spec],
        out_specs=pl.BlockSpec((2, Co), lambda b, q: (0, 0)),
        out_shape=jax.ShapeDtypeStruct((2, Co), jnp.float32),
        compiler_params=pltpu.CompilerParams(
            dimension_semantics=("arbitrary", "arbitrary")),
    )(nbr3, xrows, wft)
    W = 3 * Co + opad
    out = pl.pallas_call(
        functools.partial(_apply_body, C, Co, QB, K, F, cnt, opad),
        grid=grid,
        in_specs=[nbr_spec, x_spec, w_spec, w_spec,
                  v_spec, v_spec, pl.BlockSpec((2, Co), lambda b, q: (0, 0))],
        out_specs=pl.BlockSpec((1, QB, W), lambda b, q: (b, q, 0)),
        out_shape=jax.ShapeDtypeStruct((B, N, W), jnp.float32),
        compiler_params=pltpu.CompilerParams(
            dimension_semantics=("arbitrary", "arbitrary")),
    )(nbr3, xrows, wft, wdt, g2, b2, st)
    return out


def _fps_body(B, N, M, coor_ref, idx_ref, nc_ref):
    c0 = coor_ref[0]
    c1 = coor_ref[1]
    c2 = coor_ref[2]
    lane_n = jax.lax.broadcasted_iota(jnp.int32, (B, N), 1)
    lane_m = jax.lax.broadcasted_iota(jnp.int32, (B, M), 1)
    row_n = jax.lax.broadcasted_iota(jnp.int32, (B, N), 0)
    row_m = jax.lax.broadcasted_iota(jnp.int32, (B, M), 0)

    def body(i, carry):
        dists, acc, n0, n1, n2 = carry
        mx = jnp.max(dists, axis=1, keepdims=True)
        am = jnp.min(jnp.where(dists == mx, lane_n, N), axis=1, keepdims=True)
        far = jnp.where(i == 0, 0, am)
        mask = lane_n == far
        e0 = jnp.sum(jnp.where(mask, c0, 0.0), axis=1, keepdims=True)
        e1 = jnp.sum(jnp.where(mask, c1, 0.0), axis=1, keepdims=True)
        e2 = jnp.sum(jnp.where(mask, c2, 0.0), axis=1, keepdims=True)
        sel = lane_m == i
        acc = jnp.where(sel, jnp.broadcast_to(far, (B, M)), acc)
        n0 = jnp.where(sel, jnp.broadcast_to(e0, (B, M)), n0)
        n1 = jnp.where(sel, jnp.broadcast_to(e1, (B, M)), n1)
        n2 = jnp.where(sel, jnp.broadcast_to(e2, (B, M)), n2)
        d = (c0 - e0) ** 2 + (c1 - e1) ** 2 + (c2 - e2) ** 2
        dists = jnp.minimum(dists, d)
        return dists, acc, n0, n1, n2

    garbage = lane_m + row_m
    fgarbage = garbage.astype(jnp.float32)
    init = (jnp.where(lane_n + row_n < 0, 0.0, 1e10),
            garbage, fgarbage, fgarbage, fgarbage)
    _, acc, n0, n1, n2 = jax.lax.fori_loop(0, M, body, init)
    idx_ref[...] = acc + row_m * N
    nc_ref[0] = n0
    nc_ref[1] = n1
    nc_ref[2] = n2


def fps_call(coor3, M):
    _, B, N = coor3.shape
    return pl.pallas_call(
        functools.partial(_fps_body, B, N, M),
        in_specs=[pl.BlockSpec((3, B, N), lambda: (0, 0, 0))],
        out_specs=[pl.BlockSpec((B, M), lambda: (0, 0)),
                   pl.BlockSpec((3, B, M), lambda: (0, 0, 0))],
        out_shape=[jax.ShapeDtypeStruct((B, M), jnp.int32),
                   jax.ShapeDtypeStruct((3, B, M), jnp.float32)],
    )(coor3)


def kernel(x, W1f, W1d, g1, b1, W4f, W4d, g4, b4, W5f, W5d, g5, b5,
           W6f, W6d, g6, b6):
    B, _, N1 = x.shape
    K = KNN
    xrows = x.transpose(0, 2, 1)
    coor3 = x.transpose(1, 0, 2)

    idx1 = knn_call(xrows, QB=512)
    xpad = jnp.pad(xrows.reshape(B * N1, 3), ((0, 0), (0, 125)))
    nbr1 = sc_gather(xpad, idx1.reshape(-1))
    out1 = vn_block(nbr1, xrows, W1f, W1d, g1, b1, C=1, Co=32, QB=256,
                    opad=32)

    idxf1, ncoor1 = fps_call(coor3, 512)
    fq1 = sc_gather(out1.reshape(B * N1, 128), idxf1.reshape(-1))
    fq1 = fq1.reshape(B, 512, 128)

    idx2 = knn_call(fq1, QB=512)
    nbr2 = sc_gather(fq1.reshape(B * 512, 128), idx2.reshape(-1))
    out2 = vn_block(nbr2, fq1, W4f, W4d, g4, b4, C=32, Co=64, QB=128,
                    opad=64)

    idx3 = knn_call(out2, QB=512)
    nbr3 = sc_gather(out2.reshape(B * 512, 256), idx3.reshape(-1))
    out3 = vn_block(nbr3, out2, W5f, W5d, g5, b5, C=64, Co=64, QB=128,
                    opad=64)

    idxf2, ncoor2 = fps_call(ncoor1, 128)
    fq2 = sc_gather(out3.reshape(B * 512, 256), idxf2.reshape(-1))
    fq2 = fq2.reshape(B, 128, 256)

    idx4 = knn_call(fq2, QB=128)
    nbr4 = sc_gather(fq2.reshape(B * 128, 256), idx4.reshape(-1))
    out4 = vn_block(nbr4, fq2, W6f, W6d, g6, b6, C=64, Co=128, QB=128)

    coor_out = ncoor2.transpose(1, 0, 2)
    f_out = out4.reshape(B, 128, 3, 128).transpose(0, 3, 2, 1)
    return coor_out, f_out
